# Initial kernel scaffold; baseline (speedup 1.0000x reference)
import jax
import jax.numpy as jnp
from jax.experimental import pallas as pl


def kernel(src, lens, table, W_ih, W_hh, b_ih, b_hh):
    raise NotImplementedError("write your pallas kernel here")


# speedup vs baseline: 2.3822x; 2.3822x over previous
import functools

import jax
import jax.numpy as jnp
from jax import lax
from jax.experimental import pallas as pl
from jax.experimental.pallas import tpu as pltpu
from jax.experimental.pallas import tpu_sc as plsc


_CHUNK = 80


def _sc_gather(table, idx_flat):
    V, E = table.shape
    N = idx_flat.shape[0]
    info = plsc.get_sparse_core_info()
    NC, NS = info.num_cores, info.num_subcores
    NW = NC * NS
    assert N % (NW * _CHUNK) == 0
    n_per_w = N // NW
    n_chunks = n_per_w // _CHUNK
    idx3 = idx_flat.reshape(NW, n_chunks, _CHUNK)

    mesh = plsc.VectorSubcoreMesh(core_axis_name="c", subcore_axis_name="s")

    @functools.partial(
        pl.kernel,
        mesh=mesh,
        out_type=jax.ShapeDtypeStruct((N, E), jnp.float32),
        scratch_types=[
            pltpu.VMEM((n_chunks, _CHUNK), jnp.int32),
            pltpu.VMEM((n_per_w, E), jnp.float32),
            pltpu.SemaphoreType.DMA,
        ],
        compiler_params=pltpu.CompilerParams(use_tc_tiling_on_sc=False),
    )
    def gather_kernel(table_hbm, idx_hbm, out_hbm, idx_v, rows_v, sem):
        wid = lax.axis_index("s") * NC + lax.axis_index("c")
        pltpu.sync_copy(idx_hbm.at[wid], idx_v)
        copies = [
            pltpu.async_copy(
                table_hbm.at[idx_v.at[j]],
                rows_v.at[pl.ds(j * _CHUNK, _CHUNK)],
                sem,
            )
            for j in range(n_chunks)
        ]
        for c in copies:
            c.wait()
        pltpu.sync_copy(rows_v, out_hbm.at[pl.ds(wid * n_per_w, n_per_w)])

    return gather_kernel(table, idx3)


def _rnn_step(emb_ref, pm_ref, lens_ref, wih_ref, whh_ref, b_ref,
              out_ref, hid_ref, h_ref, *, L):
    t = pl.program_id(0)

    @pl.when(t == 0)
    def _init():
        h_ref[...] = jnp.zeros_like(h_ref)

    x = emb_ref[0] * pm_ref[0]
    h = h_ref[...]
    acc = jnp.dot(x, wih_ref[...], preferred_element_type=jnp.float32)
    acc = acc + jnp.dot(h, whh_ref[...], preferred_element_type=jnp.float32)
    h_new = jnp.tanh(acc + b_ref[...])
    valid = t < lens_ref[...]
    h_next = jnp.where(valid, h_new, h)
    h_ref[...] = h_next
    out_ref[:, 0, 0, :] = jnp.where(valid, h_new, 0.0)

    @pl.when(t == L - 1)
    def _fin():
        hid_ref[...] = h_next


def _tc_rnn(emb, pmask, lens2, wih_t, whh_t, bias, *, interpret=False):
    L, B, E = emb.shape
    H = whh_t.shape[0]
    grid = (L,)
    out_shapes = (
        jax.ShapeDtypeStruct((B, L, 1, H), jnp.float32),
        jax.ShapeDtypeStruct((B, H), jnp.float32),
    )
    return pl.pallas_call(
        functools.partial(_rnn_step, L=L),
        grid=grid,
        in_specs=[
            pl.BlockSpec((1, B, E), lambda t: (t, 0, 0)),
            pl.BlockSpec((1, B, 1), lambda t: (t, 0, 0)),
            pl.BlockSpec((B, 1), lambda t: (0, 0)),
            pl.BlockSpec((E, H), lambda t: (0, 0)),
            pl.BlockSpec((H, H), lambda t: (0, 0)),
            pl.BlockSpec((1, H), lambda t: (0, 0)),
        ],
        out_specs=(
            pl.BlockSpec((B, 1, 1, H), lambda t: (0, t, 0, 0)),
            pl.BlockSpec((B, H), lambda t: (0, 0)),
        ),
        out_shape=out_shapes,
        scratch_shapes=[pltpu.VMEM((B, H), jnp.float32)],
        compiler_params=pltpu.CompilerParams(
            dimension_semantics=("arbitrary",),
        ),
        interpret=interpret,
    )(emb, pmask, lens2, wih_t, whh_t, bias)


def kernel(src, lens, table, W_ih, W_hh, b_ih, b_hh):
    B, L = src.shape
    V, E = table.shape
    H = W_hh.shape[0]

    idx_flat = src.T.reshape(-1)
    emb_flat = _sc_gather(table, idx_flat)
    emb = emb_flat.reshape(L, B, E)

    pmask = (idx_flat != 0).astype(jnp.float32).reshape(L, B, 1)
    lens2 = lens.astype(jnp.int32).reshape(B, 1)
    bias = (b_ih + b_hh).reshape(1, H)

    out, hT = _tc_rnn(emb, pmask, lens2, W_ih.T, W_hh.T, bias)
    return out.reshape(B, L, H), hT[None]
